# baseline (speedup 1.0000x reference)
import functools

import jax
import jax.numpy as jnp
from jax import lax
import jax.experimental.pallas as pl
from jax.experimental.pallas import tpu as pltpu
from jax.experimental.pallas import tpu_sc as plsc

N = 10000
E = 320000
D = 128
G = 128
NW = 32
GPW = 81
NGP = NW * GPW
EP = NGP * G
RPT = 624

_mesh = plsc.VectorSubcoreMesh(core_axis_name="c", subcore_axis_name="s")


@functools.partial(
    pl.kernel,
    out_type=jax.ShapeDtypeStruct((2, N, D), jnp.float32),
    mesh=_mesh,
    scratch_types=[
        pltpu.VMEM((1, 3, G), jnp.int32),
        pltpu.VMEM((1, 3, G), jnp.int32),
        pltpu.VMEM((1, 3, G), jnp.int32),
        pltpu.VMEM((G, D), jnp.float32),
        pltpu.VMEM((G, D), jnp.float32),
        pltpu.VMEM((G, D), jnp.float32),
        pltpu.VMEM_SHARED((N, D), jnp.float32),
        pltpu.SemaphoreType.DMA,
        pltpu.SemaphoreType.DMA,
        pltpu.SemaphoreType.DMA,
        pltpu.SemaphoreType.DMA,
        pltpu.SemaphoreType.DMA,
        pltpu.SemaphoreType.DMA,
    ],
    compiler_params=pltpu.CompilerParams(needs_layout_passes=False),
)
def _sc_scatter(feat_h, eidx_h, out_h,
                sb0, sb1, sb2, rows0, rows1, rows2, acc,
                isem0, isem1, isem2, gsem0, gsem1, gsem2):
    c = lax.axis_index("c")
    s = lax.axis_index("s")
    w = c * 16 + s

    zv = jnp.zeros((16,), jnp.float32)

    def zrow(r, carry):
        for j in range(8):
            rows0[r, pl.ds(j * 16, 16)] = zv
        return carry

    lax.fori_loop(0, G, zrow, 0)
    r0 = s * RPT
    for i in range(4):
        pltpu.sync_copy(rows0, acc.at[pl.ds(r0 + i * G, G)])
    pltpu.sync_copy(rows0.at[pl.ds(0, RPT - 4 * G)],
                    acc.at[pl.ds(r0 + 4 * G, RPT - 4 * G)])

    @pl.when(s == 15)
    def _():
        pltpu.sync_copy(rows0.at[pl.ds(0, 16)],
                        acc.at[pl.ds(16 * RPT, 16)])

    plsc.subcore_barrier()

    gbase = w * GPW
    c0 = jnp.full((16,), 0, jnp.int32)
    c2 = jnp.full((16,), 2, jnp.int32)

    def stage(i, sb, isem):
        pltpu.async_copy(eidx_h.at[pl.ds(gbase + i, 1)], sb, isem)

    def stage_wait(i, sb, isem):
        pltpu.make_async_copy(eidx_h.at[pl.ds(gbase + i, 1)], sb, isem).wait()

    def gissue(sb, rows, gsem):
        pltpu.async_copy(feat_h.at[sb.at[0, 0]], rows, gsem)

    def gwait(sb, rows, gsem):
        pltpu.make_async_copy(feat_h.at[sb.at[0, 0]], rows, gsem).wait()

    def scale_scatter(sb, rows):
        @plsc.parallel_loop(0, G, 1, unroll=4)
        def _(e):
            vv = plsc.bitcast(
                plsc.load_gather(sb, [c0, c2, jnp.full((16,), e, jnp.int32)]),
                jnp.float32)
            for j in range(8):
                sl = pl.ds(j * 16, 16)
                rows[e, sl] = rows[e, sl] * vv

        pltpu.sync_copy(rows, acc.at[sb.at[0, 1]], add=True)

    sets = ((sb0, rows0, isem0, gsem0),
            (sb1, rows1, isem1, gsem1),
            (sb2, rows2, isem2, gsem2))

    stage(0, sb0, isem0)
    stage(1, sb1, isem1)
    stage_wait(0, sb0, isem0)
    gissue(sb0, rows0, gsem0)
    stage_wait(1, sb1, isem1)
    gissue(sb1, rows1, gsem1)

    def triple_body(p, carry):
        i0 = 3 * p
        for k in range(3):
            sbk, rowsk, _, gsemk = sets[k]
            sbn, rowsn, isemn, gsemn = sets[(k + 2) % 3]

            @pl.when(i0 + k + 2 < GPW)
            def _(i=i0 + k + 2, sbn=sbn, rowsn=rowsn,
                  isemn=isemn, gsemn=gsemn):
                stage(i, sbn, isemn)
                stage_wait(i, sbn, isemn)
                gissue(sbn, rowsn, gsemn)

            gwait(sbk, rowsk, gsemk)
            scale_scatter(sbk, rowsk)
        return carry

    lax.fori_loop(0, GPW // 3, triple_body, 0)
    plsc.subcore_barrier()

    pltpu.sync_copy(acc.at[pl.ds(r0, RPT)], out_h.at[c].at[pl.ds(r0, RPT)])

    @pl.when(s == 15)
    def _():
        pltpu.sync_copy(acc.at[pl.ds(16 * RPT, 16)],
                        out_h.at[c].at[pl.ds(16 * RPT, 16)])


_BLK = 2000
_NBLK = N // _BLK


def _tc1_body(f_ref, x0_ref, x1_ref, w1_ref, w2_ref, b1_ref, b2_ref,
              y_ref, s_ref, q_ref):
    x = x0_ref[...] + x1_ref[...]
    f = f_ref[...]
    y = jnp.dot(f + x, w1_ref[...], preferred_element_type=jnp.float32)
    y = y + jnp.dot(f * x, w2_ref[...], preferred_element_type=jnp.float32)
    y = y + b1_ref[...] + b2_ref[...]
    y_ref[...] = y

    @pl.when(pl.program_id(0) == 0)
    def _():
        s_ref[...] = jnp.zeros_like(s_ref)
        q_ref[...] = jnp.zeros_like(q_ref)

    s_ref[...] += jnp.sum(y, axis=0, keepdims=True)
    q_ref[...] += jnp.sum(y * y, axis=0, keepdims=True)


def _tc2_body(y_ref, s_ref, q_ref, g_ref, bt_ref, o_ref):
    mean = s_ref[...] * (1.0 / N)
    var = q_ref[...] * (1.0 / N) - mean * mean
    scale = lax.rsqrt(var + 1e-5) * g_ref[...]
    o_ref[...] = (y_ref[...] - mean) * scale + bt_ref[...]


def kernel(features, edge_vals, W1, b1, W2, b2, gamma, beta, edge_src, edge_dst):
    pad = EP - E
    zi = jnp.arange(pad, dtype=jnp.int32) % N
    src2d = jnp.concatenate([edge_src, zi]).reshape(NGP, 1, G)
    dst2d = jnp.concatenate([edge_dst, zi]).reshape(NGP, 1, G)
    vbits = lax.bitcast_convert_type(
        jnp.concatenate([edge_vals, jnp.zeros((pad,), jnp.float32)]),
        jnp.int32).reshape(NGP, 1, G)
    eidx = jnp.concatenate([src2d, dst2d, vbits], axis=1)

    xp = _sc_scatter(features, eidx)

    row_spec = pl.BlockSpec((_BLK, D), lambda i: (i, 0))
    full_spec = pl.BlockSpec((D, D), lambda i: (0, 0))
    vec_spec = pl.BlockSpec((1, D), lambda i: (0, 0))

    y, ssum, ssq = pl.pallas_call(
        _tc1_body,
        grid=(_NBLK,),
        in_specs=[row_spec, row_spec, row_spec, full_spec, full_spec,
                  vec_spec, vec_spec],
        out_specs=[row_spec, vec_spec, vec_spec],
        out_shape=[
            jax.ShapeDtypeStruct((N, D), jnp.float32),
            jax.ShapeDtypeStruct((1, D), jnp.float32),
            jax.ShapeDtypeStruct((1, D), jnp.float32),
        ],
        compiler_params=pltpu.CompilerParams(
            dimension_semantics=("arbitrary",)),
    )(features, xp[0], xp[1], W1, W2, b1.reshape(1, D), b2.reshape(1, D))

    out = pl.pallas_call(
        _tc2_body,
        grid=(_NBLK,),
        in_specs=[row_spec, vec_spec, vec_spec, vec_spec, vec_spec],
        out_specs=row_spec,
        out_shape=jax.ShapeDtypeStruct((N, D), jnp.float32),
        compiler_params=pltpu.CompilerParams(
            dimension_semantics=("arbitrary",)),
    )(y, ssum, ssq, gamma.reshape(1, D), beta.reshape(1, D))
    return out


# speedup vs baseline: 1.0345x; 1.0345x over previous
import functools

import jax
import jax.numpy as jnp
from jax import lax
import jax.experimental.pallas as pl
from jax.experimental.pallas import tpu as pltpu
from jax.experimental.pallas import tpu_sc as plsc

N = 10000
E = 320000
D = 128
G = 128
NW = 32
GPW = 80
NGP = NW * GPW
EP = NGP * G
RPT = 624

_mesh = plsc.VectorSubcoreMesh(core_axis_name="c", subcore_axis_name="s")


@functools.partial(
    pl.kernel,
    out_type=jax.ShapeDtypeStruct((2, N, D), jnp.float32),
    mesh=_mesh,
    scratch_types=[
        pltpu.VMEM((1, 3, G), jnp.int32),
        pltpu.VMEM((1, 3, G), jnp.int32),
        pltpu.VMEM((G, D), jnp.float32),
        pltpu.VMEM((G, D), jnp.float32),
        pltpu.VMEM_SHARED((N, D), jnp.float32),
        pltpu.SemaphoreType.DMA,
        pltpu.SemaphoreType.DMA,
        pltpu.SemaphoreType.DMA,
        pltpu.SemaphoreType.DMA,
    ],
    compiler_params=pltpu.CompilerParams(needs_layout_passes=False),
)
def _sc_scatter(feat_h, eidx_h, out_h,
                sb0, sb1, rows0, rows1, acc, isem0, isem1, gsem0, gsem1):
    c = lax.axis_index("c")
    s = lax.axis_index("s")
    w = c * 16 + s

    zv = jnp.zeros((16,), jnp.float32)

    def zrow(r, carry):
        for j in range(8):
            rows0[r, pl.ds(j * 16, 16)] = zv
        return carry

    lax.fori_loop(0, G, zrow, 0)
    r0 = s * RPT
    for i in range(4):
        pltpu.sync_copy(rows0, acc.at[pl.ds(r0 + i * G, G)])
    pltpu.sync_copy(rows0.at[pl.ds(0, RPT - 4 * G)],
                    acc.at[pl.ds(r0 + 4 * G, RPT - 4 * G)])

    @pl.when(s == 15)
    def _():
        pltpu.sync_copy(rows0.at[pl.ds(0, 16)],
                        acc.at[pl.ds(16 * RPT, 16)])

    plsc.subcore_barrier()

    gbase = w * GPW
    c0 = jnp.full((16,), 0, jnp.int32)
    c2 = jnp.full((16,), 2, jnp.int32)

    def stage(i, sb, isem):
        pltpu.async_copy(eidx_h.at[pl.ds(gbase + i, 1)], sb, isem)

    def stage_wait(i, sb, isem):
        pltpu.make_async_copy(eidx_h.at[pl.ds(gbase + i, 1)], sb, isem).wait()

    def gissue(sb, rows, gsem):
        pltpu.async_copy(feat_h.at[sb.at[0, 0]], rows, gsem)

    def gwait(sb, rows, gsem):
        pltpu.make_async_copy(feat_h.at[sb.at[0, 0]], rows, gsem).wait()

    def scale_scatter(sb, rows):
        @plsc.parallel_loop(0, G, 1, unroll=4)
        def _(e):
            vv = plsc.bitcast(
                plsc.load_gather(sb, [c0, c2, jnp.full((16,), e, jnp.int32)]),
                jnp.float32)
            for j in range(8):
                sl = pl.ds(j * 16, 16)
                rows[e, sl] = rows[e, sl] * vv

        pltpu.sync_copy(rows, acc.at[sb.at[0, 1]], add=True)

    stage(0, sb0, isem0)
    stage_wait(0, sb0, isem0)
    gissue(sb0, rows0, gsem0)
    stage(1, sb1, isem1)

    def pair_body(p, carry):
        i0 = 2 * p
        stage_wait(i0 + 1, sb1, isem1)
        gissue(sb1, rows1, gsem1)
        gwait(sb0, rows0, gsem0)
        scale_scatter(sb0, rows0)

        @pl.when(i0 + 2 < GPW)
        def _():
            stage(i0 + 2, sb0, isem0)
            stage_wait(i0 + 2, sb0, isem0)
            gissue(sb0, rows0, gsem0)

        gwait(sb1, rows1, gsem1)
        scale_scatter(sb1, rows1)

        @pl.when(i0 + 3 < GPW)
        def _():
            stage(i0 + 3, sb1, isem1)

        return carry

    lax.fori_loop(0, GPW // 2, pair_body, 0)
    plsc.subcore_barrier()

    pltpu.sync_copy(acc.at[pl.ds(r0, RPT)], out_h.at[c].at[pl.ds(r0, RPT)])

    @pl.when(s == 15)
    def _():
        pltpu.sync_copy(acc.at[pl.ds(16 * RPT, 16)],
                        out_h.at[c].at[pl.ds(16 * RPT, 16)])


_BLK = 2000
_NBLK = N // _BLK


def _tc_body(f_ref, x0_ref, x1_ref, w1_ref, w2_ref, b1_ref, b2_ref,
             g_ref, bt_ref, o_ref, y_s, s_s, q_s):
    i = pl.program_id(0)

    @pl.when(i < _NBLK)
    def _():
        x = x0_ref[...] + x1_ref[...]
        f = f_ref[...]
        y = jnp.dot(f + x, w1_ref[...], preferred_element_type=jnp.float32)
        y = y + jnp.dot(f * x, w2_ref[...], preferred_element_type=jnp.float32)
        y = y + b1_ref[...] + b2_ref[...]
        y_s[pl.ds(i * _BLK, _BLK), :] = y

        @pl.when(i == 0)
        def _():
            s_s[...] = jnp.zeros_like(s_s)
            q_s[...] = jnp.zeros_like(q_s)

        s_s[...] += jnp.sum(y, axis=0, keepdims=True)
        q_s[...] += jnp.sum(y * y, axis=0, keepdims=True)

    @pl.when(i >= _NBLK)
    def _():
        j = i - _NBLK
        mean = s_s[...] * (1.0 / N)
        var = q_s[...] * (1.0 / N) - mean * mean
        scale = lax.rsqrt(var + 1e-5) * g_ref[...]
        o_ref[...] = (y_s[pl.ds(j * _BLK, _BLK), :] - mean) * scale \
            + bt_ref[...]


def kernel(features, edge_vals, W1, b1, W2, b2, gamma, beta, edge_src, edge_dst):
    pad = EP - E
    zi = jnp.arange(pad, dtype=jnp.int32) % N
    src2d = jnp.concatenate([edge_src, zi]).reshape(NGP, 1, G)
    dst2d = jnp.concatenate([edge_dst, zi]).reshape(NGP, 1, G)
    vbits = lax.bitcast_convert_type(
        jnp.concatenate([edge_vals, jnp.zeros((pad,), jnp.float32)]),
        jnp.int32).reshape(NGP, 1, G)
    eidx = jnp.concatenate([src2d, dst2d, vbits], axis=1)

    xp = _sc_scatter(features, eidx)

    in_row_spec = pl.BlockSpec(
        (_BLK, D), lambda i: (jnp.minimum(i, _NBLK - 1), 0))
    out_row_spec = pl.BlockSpec(
        (_BLK, D), lambda i: (jnp.maximum(i - _NBLK, 0), 0))
    full_spec = pl.BlockSpec((D, D), lambda i: (0, 0))
    vec_spec = pl.BlockSpec((1, D), lambda i: (0, 0))

    out = pl.pallas_call(
        _tc_body,
        grid=(2 * _NBLK,),
        in_specs=[in_row_spec, in_row_spec, in_row_spec, full_spec,
                  full_spec, vec_spec, vec_spec, vec_spec, vec_spec],
        out_specs=out_row_spec,
        out_shape=jax.ShapeDtypeStruct((N, D), jnp.float32),
        scratch_shapes=[
            pltpu.VMEM((N, D), jnp.float32),
            pltpu.VMEM((1, D), jnp.float32),
            pltpu.VMEM((1, D), jnp.float32),
        ],
        compiler_params=pltpu.CompilerParams(
            dimension_semantics=("arbitrary",)),
    )(features, xp[0], xp[1], W1, W2, b1.reshape(1, D), b2.reshape(1, D),
      gamma.reshape(1, D), beta.reshape(1, D))
    return out
